# Initial kernel scaffold; baseline (speedup 1.0000x reference)
import jax
import jax.numpy as jnp
from jax.experimental import pallas as pl


def kernel(x, edge_index, W1_l, W1_r, b1, W2_l, W2_r, b2):
    raise NotImplementedError("write your pallas kernel here")


# speedup vs baseline: 5.0554x; 5.0554x over previous
import functools

import jax
import jax.numpy as jnp
from jax import lax
from jax.experimental import pallas as pl
from jax.experimental.pallas import tpu as pltpu
from jax.experimental.pallas import tpu_sc as plsc

N_CORES = 2
N_SUBCORES = 16
N_TILES = N_CORES * N_SUBCORES
CHUNK = 128


def _rows_acc(n_nodes):
    step = N_SUBCORES * 8
    return ((n_nodes + 1 + step - 1) // step) * step


def _aggregate_body(n_chunks, rows_per_tile, feat, srcp, dstp, zeros_f,
                    out_sum, srcv, dstv, rowsv, acc, sem):
    cid = lax.axis_index("c")
    sid = lax.axis_index("s")
    wid = cid * N_SUBCORES + sid

    base = sid * rows_per_tile
    pltpu.sync_copy(zeros_f, acc.at[pl.ds(base, rows_per_tile)])
    pltpu.sync_copy(srcp.at[wid], srcv)
    pltpu.sync_copy(dstp.at[wid], dstv)
    plsc.subcore_barrier()

    def chunk_step(j, carry):
        pltpu.async_copy(feat.at[srcv.at[j]], rowsv, sem).wait()
        pltpu.sync_copy(rowsv, acc.at[dstv.at[j]], add=True)
        return carry

    lax.fori_loop(0, n_chunks, chunk_step, 0)
    plsc.subcore_barrier()

    pltpu.sync_copy(acc.at[pl.ds(base, rows_per_tile)],
                    out_sum.at[cid, pl.ds(base, rows_per_tile)])


def _make_aggregate(n_nodes, d, n_chunks):
    rows = _rows_acc(n_nodes)
    rows_per_tile = rows // N_SUBCORES
    mesh = plsc.VectorSubcoreMesh(core_axis_name="c", subcore_axis_name="s")
    out_type = jax.ShapeDtypeStruct((N_CORES, rows, d), jnp.float32)
    scratch = [
        pltpu.VMEM((n_chunks, CHUNK), jnp.int32),
        pltpu.VMEM((n_chunks, CHUNK), jnp.int32),
        pltpu.VMEM((CHUNK, d), jnp.float32),
        pltpu.VMEM_SHARED((rows, d), jnp.float32),
        pltpu.SemaphoreType.DMA,
    ]
    body = functools.partial(_aggregate_body, n_chunks, rows_per_tile)
    return pl.kernel(body, out_type=out_type, mesh=mesh,
                     scratch_types=scratch)


def _count_body(n_chunks, rows_per_tile, d, dstp, zeros_f, ones_h, out_cnt,
                dstv, onesv, cntacc):
    cid = lax.axis_index("c")
    sid = lax.axis_index("s")
    wid = cid * N_SUBCORES + sid

    base = sid * rows_per_tile
    pltpu.sync_copy(zeros_f, cntacc.at[pl.ds(base, rows_per_tile)])
    pltpu.sync_copy(ones_h, onesv)
    pltpu.sync_copy(dstp.at[wid], dstv)
    plsc.subcore_barrier()

    def chunk_step(j, carry):
        pltpu.sync_copy(onesv, cntacc.at[dstv.at[j]], add=True)
        return carry

    lax.fori_loop(0, n_chunks, chunk_step, 0)
    plsc.subcore_barrier()

    pltpu.sync_copy(cntacc.at[pl.ds(base, rows_per_tile)],
                    out_cnt.at[cid, pl.ds(base, rows_per_tile)])


def _make_count(n_nodes, d, n_chunks):
    rows = _rows_acc(n_nodes)
    rows_per_tile = rows // N_SUBCORES
    mesh = plsc.VectorSubcoreMesh(core_axis_name="c", subcore_axis_name="s")
    out_type = jax.ShapeDtypeStruct((N_CORES, rows, d), jnp.float32)
    scratch = [
        pltpu.VMEM((n_chunks, CHUNK), jnp.int32),
        pltpu.VMEM((CHUNK, d), jnp.float32),
        pltpu.VMEM_SHARED((rows, d), jnp.float32),
    ]
    body = functools.partial(_count_body, n_chunks, rows_per_tile, d)
    return pl.kernel(body, out_type=out_type, mesh=mesh,
                     scratch_types=scratch)


def _dense_body(n_nodes, relu, p_ref, c_ref, x_ref, wl_ref, wr_ref, b_ref, o_ref):
    s = p_ref[0, :n_nodes, :] + p_ref[1, :n_nodes, :]
    cnt = c_ref[0, :n_nodes, 0] + c_ref[1, :n_nodes, 0]
    mean = s / jnp.maximum(cnt, 1.0)[:, None]
    dn = (((1,), (1,)), ((), ()))
    out = (lax.dot_general(mean, wl_ref[...], dn, preferred_element_type=jnp.float32)
           + lax.dot_general(x_ref[...], wr_ref[...], dn, preferred_element_type=jnp.float32)
           + b_ref[...])
    o_ref[...] = jnp.maximum(out, 0.0) if relu else out


def _dense(p, cnt, x, w_l, w_r, b, relu):
    n_nodes, d = x.shape
    return pl.pallas_call(
        functools.partial(_dense_body, n_nodes, relu),
        out_shape=jax.ShapeDtypeStruct((n_nodes, d), jnp.float32),
    )(p, cnt, x, w_l, w_r, b.reshape(1, -1))


def kernel(x, edge_index, W1_l, W1_r, b1, W2_l, W2_r, b2):
    n_nodes, d = x.shape
    e = edge_index.shape[1]
    src = edge_index[0].astype(jnp.int32)
    dst = edge_index[1].astype(jnp.int32)

    n_chunks = -(-e // (N_TILES * CHUNK))
    e_pad = N_TILES * n_chunks * CHUNK
    srcp = jnp.concatenate([src, jnp.zeros((e_pad - e,), jnp.int32)])
    dstp = jnp.concatenate([dst, jnp.full((e_pad - e,), n_nodes, jnp.int32)])
    srcp = srcp.reshape(N_TILES, n_chunks, CHUNK)
    dstp = dstp.reshape(N_TILES, n_chunks, CHUNK)

    agg = _make_aggregate(n_nodes, d, n_chunks)
    count = _make_count(n_nodes, d, n_chunks)
    rows_per_tile = _rows_acc(n_nodes) // N_SUBCORES
    zeros_f = jnp.zeros((rows_per_tile, d), jnp.float32)
    ones_h = jnp.ones((CHUNK, d), jnp.float32)

    cnt = count(dstp, zeros_f, ones_h)
    p1 = agg(x, srcp, dstp, zeros_f)
    h = _dense(p1, cnt, x, W1_l, W1_r, b1, relu=True)
    p2 = agg(h, srcp, dstp, zeros_f)
    return _dense(p2, cnt, h, W2_l, W2_r, b2, relu=False)
